# baseline (speedup 1.0000x reference)
import functools

import jax
import jax.numpy as jnp
from jax import lax
from jax.experimental import pallas as pl
from jax.experimental.pallas import tpu as pltpu
from jax.experimental.pallas import tpu_sc as plsc

N = 10000
NP = 10240
AR = 10224
E = 320000
D = 128
W = 64
NC = 2
NS = 16
NW = NC * NS
EPW = 10240
EPS = 20480
CHB = 128
NCHB = EPS // CHB
CHD = 512
NCHD = EPW // CHD
DW = 16
RPS = NP // NS
RPA = AR // NS

_MESH = plsc.VectorSubcoreMesh(core_axis_name="c", subcore_axis_name="s",
                               num_cores=NC, num_subcores=NS)
_UNTILED = pltpu.CompilerParams(use_tc_tiling_on_sc=False)


_NSEM = 4


@functools.partial(
    pl.kernel,
    out_type=jax.ShapeDtypeStruct((NC, NP, DW), jnp.float32),
    mesh=_MESH,
    compiler_params=_UNTILED,
    scratch_types=[
        pltpu.VMEM((EPW,), jnp.int32),
        pltpu.VMEM((CHD, DW), jnp.float32),
        pltpu.VMEM_SHARED((NP, DW), jnp.float32),
    ] + [pltpu.SemaphoreType.DMA] * _NSEM,
)
def _sc_degree(dst_hbm, zero_hbm, ones_hbm, out_hbm, idx_v, ones_v, acc, *sems):
    cid = lax.axis_index("c")
    sid = lax.axis_index("s")
    wid = sid * NC + cid
    pltpu.sync_copy(zero_hbm, acc.at[pl.ds(sid * RPS, RPS)])
    pltpu.sync_copy(ones_hbm, ones_v)
    pltpu.sync_copy(dst_hbm.at[wid], idx_v)
    plsc.subcore_barrier()

    def body(g, carry):
        for b in range(_NSEM):
            j = g * _NSEM + b

            @pl.when(g > 0)
            def _():
                pltpu.make_async_copy(ones_v, acc.at[pl.ds(0, CHD)],
                                      sems[b]).wait()

            pltpu.async_copy(ones_v, acc.at[idx_v.at[pl.ds(j * CHD, CHD)]],
                             sems[b], add=True)
        return carry

    lax.fori_loop(0, NCHD // _NSEM, body, 0)
    for b in range(_NSEM):
        pltpu.make_async_copy(ones_v, acc.at[pl.ds(0, CHD)], sems[b]).wait()
    plsc.subcore_barrier()
    pltpu.sync_copy(acc.at[pl.ds(sid * RPS, RPS)],
                    out_hbm.at[cid, pl.ds(sid * RPS, RPS)])


_NBUF = 6


@functools.partial(
    pl.kernel,
    out_type=jax.ShapeDtypeStruct((NC, NP, W), jnp.float32),
    mesh=_MESH,
    compiler_params=_UNTILED,
    scratch_types=[
        pltpu.VMEM((NCHB, CHB), jnp.int32),
        pltpu.VMEM((NCHB, CHB), jnp.int32),
        pltpu.VMEM((_NBUF, CHB, W), jnp.float32),
        pltpu.VMEM_SHARED((AR, W), jnp.float32),
    ] + [pltpu.SemaphoreType.DMA] * (2 * _NBUF),
)
def _sc_scatter(table_hbm, src_hbm, dst_hbm, zero_hbm, out_hbm,
                idx_s, idx_d, rows_v, acc, *sems):
    gsem = sems[:_NBUF]
    ssem = sems[_NBUF:]
    cid = lax.axis_index("c")
    sid = lax.axis_index("s")
    pltpu.sync_copy(zero_hbm.at[pl.ds(0, RPA)], acc.at[pl.ds(sid * RPA, RPA)])
    pltpu.sync_copy(src_hbm.at[sid], idx_s)
    pltpu.sync_copy(dst_hbm.at[sid], idx_d)
    plsc.subcore_barrier()

    def gather(j, b):
        pltpu.async_copy(table_hbm.at[cid].at[idx_s.at[j]],
                         rows_v.at[b], gsem[b])

    def scatter(j, b):
        pltpu.async_copy(rows_v.at[b], acc.at[idx_d.at[j]],
                         ssem[b], add=True)

    def wait_scatter(b):
        pltpu.make_async_copy(rows_v.at[b], acc.at[pl.ds(0, CHB)],
                              ssem[b]).wait()

    def wait_gather(b):
        pltpu.make_async_copy(zero_hbm.at[pl.ds(0, CHB)], rows_v.at[b],
                              gsem[b]).wait()

    for b in range(4):
        gather(b, b)

    def body(g, carry):
        for b in range(_NBUF):
            j = g * _NBUF + b
            bn = (b + 4) % _NBUF

            @pl.when(j >= 2)
            def _():
                wait_scatter(bn)

            gather(j + 4, bn)
            wait_gather(b)
            scatter(j, b)
        return carry

    lax.fori_loop(0, (NCHB - 4) // _NBUF, body, 0)
    for j in range(NCHB - 4, NCHB):
        wait_gather(j % _NBUF)
        scatter(j, j % _NBUF)
    for b in range(_NBUF):
        wait_scatter(b)
    plsc.subcore_barrier()
    pltpu.sync_copy(acc.at[pl.ds(sid * RPA, RPA)],
                    out_hbm.at[cid, pl.ds(sid * RPA, RPA)])

    @pl.when(sid == 0)
    def _():
        pltpu.sync_copy(zero_hbm.at[pl.ds(0, NP - AR)],
                        out_hbm.at[cid, pl.ds(AR, NP - AR)])


_BLK = 1280


def _prep_body(x_ref, w_ref, d0_ref, d1_ref, t_ref, dinv_ref):
    dp = d0_ref[...] + d1_ref[...]
    deg = 1.0 + jnp.sum(dp, axis=1, keepdims=True) * (1.0 / DW)
    dinv = lax.rsqrt(deg)
    xw = jnp.dot(x_ref[...], w_ref[...], preferred_element_type=jnp.float32)
    t = dinv * xw
    t_ref[0, :, :] = t[:, :W]
    t_ref[1, :, :] = t[:, W:]
    dinv_ref[...] = dinv


def _tc_prep(x, w1, d0, d1):
    return pl.pallas_call(
        _prep_body,
        grid=(NP // _BLK,),
        in_specs=[
            pl.BlockSpec((_BLK, D), lambda i: (i, 0)),
            pl.BlockSpec((D, D), lambda i: (0, 0)),
            pl.BlockSpec((_BLK, DW), lambda i: (i, 0)),
            pl.BlockSpec((_BLK, DW), lambda i: (i, 0)),
        ],
        out_specs=[
            pl.BlockSpec((2, _BLK, W), lambda i: (0, i, 0)),
            pl.BlockSpec((_BLK, 1), lambda i: (i, 0)),
        ],
        out_shape=[
            jax.ShapeDtypeStruct((2, NP, W), jnp.float32),
            jax.ShapeDtypeStruct((NP, 1), jnp.float32),
        ],
    )(x, w1, d0, d1)


def _mid_body(pa_ref, pb_ref, ta_ref, tb_ref, dinv_ref, b_ref, w_ref, out_ref):
    s = jnp.concatenate([pa_ref[0] + ta_ref[0], pb_ref[0] + tb_ref[0]],
                        axis=-1)
    dinv = dinv_ref[...]
    h = dinv * s + b_ref[...]
    h = jnp.maximum(h, 0.0)
    hw = jnp.dot(h, w_ref[...], preferred_element_type=jnp.float32)
    t = dinv * hw
    out_ref[0, :, :] = t[:, :W]
    out_ref[1, :, :] = t[:, W:]


def _tc_mid(p, t1, dinv, b1, w2):
    half = lambda k: pl.BlockSpec((1, _BLK, W), lambda i, k=k: (k, i, 0))
    return pl.pallas_call(
        _mid_body,
        grid=(NP // _BLK,),
        in_specs=[
            half(0), half(1), half(0), half(1),
            pl.BlockSpec((_BLK, 1), lambda i: (i, 0)),
            pl.BlockSpec((1, D), lambda i: (0, 0)),
            pl.BlockSpec((D, D), lambda i: (0, 0)),
        ],
        out_specs=pl.BlockSpec((2, _BLK, W), lambda i: (0, i, 0)),
        out_shape=jax.ShapeDtypeStruct((2, NP, W), jnp.float32),
    )(p, p, t1, t1, dinv, b1, w2)


def _out_body(qa_ref, qb_ref, ta_ref, tb_ref, dinv_ref, b_ref, out_ref):
    s = jnp.concatenate([qa_ref[0] + ta_ref[0], qb_ref[0] + tb_ref[0]],
                        axis=-1)
    out_ref[...] = dinv_ref[...] * s + b_ref[...]


def _tc_out(q, t2, dinv, b2):
    half = lambda k: pl.BlockSpec((1, _BLK, W), lambda i, k=k: (k, i, 0))
    return pl.pallas_call(
        _out_body,
        grid=(NP // _BLK,),
        in_specs=[
            half(0), half(1), half(0), half(1),
            pl.BlockSpec((_BLK, 1), lambda i: (i, 0)),
            pl.BlockSpec((1, D), lambda i: (0, 0)),
        ],
        out_specs=pl.BlockSpec((_BLK, D), lambda i: (i, 0)),
        out_shape=jax.ShapeDtypeStruct((NP, D), jnp.float32),
    )(q, q, t2, t2, dinv, b2)


def kernel(x, edge_index, W1, b1, W2, b2):
    ei = edge_index.astype(jnp.int32)
    padw = EPW * NW - E
    dstw = jnp.concatenate([ei[1], jnp.full((padw,), N, jnp.int32)]
                           ).reshape(NW, EPW)
    pads = EPS * NS - E
    srcs = jnp.concatenate([ei[0], jnp.zeros((pads,), jnp.int32)]
                           ).reshape(NS, NCHB, CHB)
    dsts = jnp.concatenate([ei[1], jnp.full((pads,), N, jnp.int32)]
                           ).reshape(NS, NCHB, CHB)
    x_p = jnp.pad(x, ((0, NP - N), (0, 0)))
    zero_w = jnp.zeros((RPS, W), jnp.float32)
    zero_deg = jnp.zeros((RPS, DW), jnp.float32)
    ones_deg = jnp.ones((CHD, DW), jnp.float32)

    degp = _sc_degree(dstw, zero_deg, ones_deg)
    t1, dinv = _tc_prep(x_p, W1, degp[0], degp[1])
    p = _sc_scatter(t1, srcs, dsts, zero_w)
    t2 = _tc_mid(p, t1, dinv, b1.reshape(1, D), W2)
    q = _sc_scatter(t2, srcs, dsts, zero_w)
    out = _tc_out(q, t2, dinv, b2.reshape(1, D))
    return out[:N]


# speedup vs baseline: 1.0037x; 1.0037x over previous
import functools

import jax
import jax.numpy as jnp
from jax import lax
from jax.experimental import pallas as pl
from jax.experimental.pallas import tpu as pltpu
from jax.experimental.pallas import tpu_sc as plsc

N = 10000
NP = 10240
AR = 10240
E = 320000
D = 128
W = 64
NC = 2
NS = 16
NW = NC * NS
EPW = 10240
EPS = 20480
CHB = 128
NCHB = EPS // CHB
CHD = 512
NCHD = EPW // CHD
DW = 16
RPS = NP // NS
RPA = AR // NS

_MESH = plsc.VectorSubcoreMesh(core_axis_name="c", subcore_axis_name="s",
                               num_cores=NC, num_subcores=NS)
_UNTILED = pltpu.CompilerParams(use_tc_tiling_on_sc=False)


_NSEM = 4


@functools.partial(
    pl.kernel,
    out_type=jax.ShapeDtypeStruct((NC, NP, DW), jnp.float32),
    mesh=_MESH,
    compiler_params=_UNTILED,
    scratch_types=[
        pltpu.VMEM((EPW,), jnp.int32),
        pltpu.VMEM((CHD, DW), jnp.float32),
        pltpu.VMEM_SHARED((NP, DW), jnp.float32),
    ] + [pltpu.SemaphoreType.DMA] * _NSEM,
)
def _sc_degree(dst_hbm, zero_hbm, ones_hbm, out_hbm, idx_v, ones_v, acc, *sems):
    cid = lax.axis_index("c")
    sid = lax.axis_index("s")
    wid = sid * NC + cid
    pltpu.sync_copy(zero_hbm, acc.at[pl.ds(sid * RPS, RPS)])
    pltpu.sync_copy(ones_hbm, ones_v)
    pltpu.sync_copy(dst_hbm.at[wid], idx_v)
    plsc.subcore_barrier()

    def body(g, carry):
        for b in range(_NSEM):
            j = g * _NSEM + b

            @pl.when(g > 0)
            def _():
                pltpu.make_async_copy(ones_v, acc.at[pl.ds(0, CHD)],
                                      sems[b]).wait()

            pltpu.async_copy(ones_v, acc.at[idx_v.at[pl.ds(j * CHD, CHD)]],
                             sems[b], add=True)
        return carry

    lax.fori_loop(0, NCHD // _NSEM, body, 0)
    for b in range(_NSEM):
        pltpu.make_async_copy(ones_v, acc.at[pl.ds(0, CHD)], sems[b]).wait()
    plsc.subcore_barrier()
    pltpu.sync_copy(acc.at[pl.ds(sid * RPS, RPS)],
                    out_hbm.at[cid, pl.ds(sid * RPS, RPS)])


_NBUF = 5


@functools.partial(
    pl.kernel,
    out_type=jax.ShapeDtypeStruct((NC, NP, W), jnp.float32),
    mesh=_MESH,
    compiler_params=_UNTILED,
    scratch_types=[
        pltpu.VMEM((NCHB, CHB), jnp.int32),
        pltpu.VMEM((NCHB, CHB), jnp.int32),
        pltpu.VMEM((_NBUF, CHB, W), jnp.float32),
        pltpu.VMEM_SHARED((AR, W), jnp.float32),
    ] + [pltpu.SemaphoreType.DMA] * (2 * _NBUF),
)
def _sc_scatter(table_hbm, src_hbm, dst_hbm, zero_hbm, out_hbm,
                idx_s, idx_d, rows_v, acc, *sems):
    gsem = sems[:_NBUF]
    ssem = sems[_NBUF:]
    cid = lax.axis_index("c")
    sid = lax.axis_index("s")
    pltpu.sync_copy(zero_hbm.at[pl.ds(0, RPA)], acc.at[pl.ds(sid * RPA, RPA)])
    pltpu.sync_copy(src_hbm.at[sid], idx_s)
    pltpu.sync_copy(dst_hbm.at[sid], idx_d)
    plsc.subcore_barrier()

    def gather(j, b):
        pltpu.async_copy(table_hbm.at[cid].at[idx_s.at[j]],
                         rows_v.at[b], gsem[b])

    def scatter(j, b):
        pltpu.async_copy(rows_v.at[b], acc.at[idx_d.at[j]],
                         ssem[b], add=True)

    def wait_scatter(b):
        pltpu.make_async_copy(rows_v.at[b], acc.at[pl.ds(0, CHB)],
                              ssem[b]).wait()

    def wait_gather(b):
        pltpu.make_async_copy(zero_hbm.at[pl.ds(0, CHB)], rows_v.at[b],
                              gsem[b]).wait()

    for b in range(3):
        gather(b, b)

    def body(g, carry):
        for b in range(_NBUF):
            j = g * _NBUF + b
            bn = (b + 3) % _NBUF

            @pl.when(j >= 2)
            def _():
                wait_scatter(bn)

            @pl.when(j < NCHB - 3)
            def _():
                gather(j + 3, bn)

            wait_gather(b)
            scatter(j, b)
        return carry

    lax.fori_loop(0, NCHB // _NBUF, body, 0)
    for j in (NCHB - 2, NCHB - 1):
        wait_scatter(j % _NBUF)
    plsc.subcore_barrier()
    pltpu.sync_copy(acc.at[pl.ds(sid * RPA, RPA)],
                    out_hbm.at[cid, pl.ds(sid * RPA, RPA)])


_BLK = 1280


def _prep_body(x_ref, w_ref, d0_ref, d1_ref, t_ref, dinv_ref):
    dp = d0_ref[...] + d1_ref[...]
    deg = 1.0 + jnp.sum(dp, axis=1, keepdims=True) * (1.0 / DW)
    dinv = lax.rsqrt(deg)
    xw = jnp.dot(x_ref[...], w_ref[...], preferred_element_type=jnp.float32)
    t = dinv * xw
    t_ref[0, :, :] = t[:, :W]
    t_ref[1, :, :] = t[:, W:]
    dinv_ref[...] = dinv


def _tc_prep(x, w1, d0, d1):
    return pl.pallas_call(
        _prep_body,
        grid=(NP // _BLK,),
        in_specs=[
            pl.BlockSpec((_BLK, D), lambda i: (i, 0)),
            pl.BlockSpec((D, D), lambda i: (0, 0)),
            pl.BlockSpec((_BLK, DW), lambda i: (i, 0)),
            pl.BlockSpec((_BLK, DW), lambda i: (i, 0)),
        ],
        out_specs=[
            pl.BlockSpec((2, _BLK, W), lambda i: (0, i, 0)),
            pl.BlockSpec((_BLK, 1), lambda i: (i, 0)),
        ],
        out_shape=[
            jax.ShapeDtypeStruct((2, NP, W), jnp.float32),
            jax.ShapeDtypeStruct((NP, 1), jnp.float32),
        ],
    )(x, w1, d0, d1)


def _mid_body(pa_ref, pb_ref, ta_ref, tb_ref, dinv_ref, b_ref, w_ref, out_ref):
    s = jnp.concatenate([pa_ref[0] + ta_ref[0], pb_ref[0] + tb_ref[0]],
                        axis=-1)
    dinv = dinv_ref[...]
    h = dinv * s + b_ref[...]
    h = jnp.maximum(h, 0.0)
    hw = jnp.dot(h, w_ref[...], preferred_element_type=jnp.float32)
    t = dinv * hw
    out_ref[0, :, :] = t[:, :W]
    out_ref[1, :, :] = t[:, W:]


def _tc_mid(p, t1, dinv, b1, w2):
    half = lambda k: pl.BlockSpec((1, _BLK, W), lambda i, k=k: (k, i, 0))
    return pl.pallas_call(
        _mid_body,
        grid=(NP // _BLK,),
        in_specs=[
            half(0), half(1), half(0), half(1),
            pl.BlockSpec((_BLK, 1), lambda i: (i, 0)),
            pl.BlockSpec((1, D), lambda i: (0, 0)),
            pl.BlockSpec((D, D), lambda i: (0, 0)),
        ],
        out_specs=pl.BlockSpec((2, _BLK, W), lambda i: (0, i, 0)),
        out_shape=jax.ShapeDtypeStruct((2, NP, W), jnp.float32),
    )(p, p, t1, t1, dinv, b1, w2)


def _out_body(qa_ref, qb_ref, ta_ref, tb_ref, dinv_ref, b_ref, out_ref):
    s = jnp.concatenate([qa_ref[0] + ta_ref[0], qb_ref[0] + tb_ref[0]],
                        axis=-1)
    out_ref[...] = dinv_ref[...] * s + b_ref[...]


def _tc_out(q, t2, dinv, b2):
    half = lambda k: pl.BlockSpec((1, _BLK, W), lambda i, k=k: (k, i, 0))
    return pl.pallas_call(
        _out_body,
        grid=(NP // _BLK,),
        in_specs=[
            half(0), half(1), half(0), half(1),
            pl.BlockSpec((_BLK, 1), lambda i: (i, 0)),
            pl.BlockSpec((1, D), lambda i: (0, 0)),
        ],
        out_specs=pl.BlockSpec((_BLK, D), lambda i: (i, 0)),
        out_shape=jax.ShapeDtypeStruct((NP, D), jnp.float32),
    )(q, q, t2, t2, dinv, b2)


def kernel(x, edge_index, W1, b1, W2, b2):
    ei = edge_index.astype(jnp.int32)
    padw = EPW * NW - E
    dstw = jnp.concatenate([ei[1], jnp.full((padw,), N, jnp.int32)]
                           ).reshape(NW, EPW)
    pads = EPS * NS - E
    srcs = jnp.concatenate([ei[0], jnp.zeros((pads,), jnp.int32)]
                           ).reshape(NS, NCHB, CHB)
    dsts = jnp.concatenate([ei[1], jnp.full((pads,), N, jnp.int32)]
                           ).reshape(NS, NCHB, CHB)
    x_p = jnp.pad(x, ((0, NP - N), (0, 0)))
    zero_w = jnp.zeros((RPS, W), jnp.float32)
    zero_deg = jnp.zeros((RPS, DW), jnp.float32)
    ones_deg = jnp.ones((CHD, DW), jnp.float32)

    degp = _sc_degree(dstw, zero_deg, ones_deg)
    t1, dinv = _tc_prep(x_p, W1, degp[0], degp[1])
    p = _sc_scatter(t1, srcs, dsts, zero_w)
    t2 = _tc_mid(p, t1, dinv, b1.reshape(1, D), W2)
    q = _sc_scatter(t2, srcs, dsts, zero_w)
    out = _tc_out(q, t2, dinv, b2.reshape(1, D))
    return out[:N]


# speedup vs baseline: 1.1153x; 1.1111x over previous
import functools

import jax
import jax.numpy as jnp
from jax import lax
from jax.experimental import pallas as pl
from jax.experimental.pallas import tpu as pltpu
from jax.experimental.pallas import tpu_sc as plsc

N = 10000
NP = 10240
AR = 10240
E = 320000
D = 128
W = 64
NC = 2
NS = 16
NW = NC * NS
EPW = 10240
EPS = 20480
CHB = 128
NCHB = EPS // CHB
CHD = 512
NCHD = EPW // CHD
DW = 16
RPS = NP // NS
RPA = AR // NS

_MESH = plsc.VectorSubcoreMesh(core_axis_name="c", subcore_axis_name="s",
                               num_cores=NC, num_subcores=NS)
_UNTILED = pltpu.CompilerParams(use_tc_tiling_on_sc=False)


_NSEM = 4


@functools.partial(
    pl.kernel,
    out_type=jax.ShapeDtypeStruct((NC, NP, DW), jnp.float32),
    mesh=_MESH,
    compiler_params=_UNTILED,
    scratch_types=[
        pltpu.VMEM((EPW,), jnp.int32),
        pltpu.VMEM((CHD, DW), jnp.float32),
        pltpu.VMEM_SHARED((NP, DW), jnp.float32),
    ] + [pltpu.SemaphoreType.DMA] * _NSEM,
)
def _sc_degree(dst_hbm, zero_hbm, ones_hbm, out_hbm, idx_v, ones_v, acc, *sems):
    cid = lax.axis_index("c")
    sid = lax.axis_index("s")
    wid = sid * NC + cid
    pltpu.sync_copy(zero_hbm, acc.at[pl.ds(sid * RPS, RPS)])
    pltpu.sync_copy(ones_hbm, ones_v)
    pltpu.sync_copy(dst_hbm.at[wid], idx_v)
    plsc.subcore_barrier()

    def body(g, carry):
        for b in range(_NSEM):
            j = g * _NSEM + b

            @pl.when(g > 0)
            def _():
                pltpu.make_async_copy(ones_v, acc.at[pl.ds(0, CHD)],
                                      sems[b]).wait()

            pltpu.async_copy(ones_v, acc.at[idx_v.at[pl.ds(j * CHD, CHD)]],
                             sems[b], add=True)
        return carry

    lax.fori_loop(0, NCHD // _NSEM, body, 0)
    for b in range(_NSEM):
        pltpu.make_async_copy(ones_v, acc.at[pl.ds(0, CHD)], sems[b]).wait()
    plsc.subcore_barrier()
    pltpu.sync_copy(acc.at[pl.ds(sid * RPS, RPS)],
                    out_hbm.at[cid, pl.ds(sid * RPS, RPS)])


_NBUF = 5


@functools.partial(
    pl.kernel,
    out_type=jax.ShapeDtypeStruct((NC, NP, W), jnp.float32),
    mesh=_MESH,
    compiler_params=_UNTILED,
    scratch_types=[
        pltpu.VMEM((NCHB, CHB), jnp.int32),
        pltpu.VMEM((NCHB, CHB), jnp.int32),
        pltpu.VMEM((_NBUF, CHB, W), jnp.float32),
        pltpu.VMEM_SHARED((AR, W), jnp.float32),
    ] + [pltpu.SemaphoreType.DMA] * (2 * _NBUF),
)
def _sc_scatter(table_hbm, src_hbm, dst_hbm, zero_hbm, out_hbm,
                idx_s, idx_d, rows_v, acc, *sems):
    gsem = sems[:_NBUF]
    ssem = sems[_NBUF:]
    cid = lax.axis_index("c")
    sid = lax.axis_index("s")
    pltpu.sync_copy(zero_hbm.at[pl.ds(0, RPA)], acc.at[pl.ds(sid * RPA, RPA)])
    pltpu.sync_copy(src_hbm.at[sid], idx_s)
    pltpu.sync_copy(dst_hbm.at[sid], idx_d)
    plsc.subcore_barrier()

    def gather(j, b):
        pltpu.async_copy(table_hbm.at[cid].at[idx_s.at[j]],
                         rows_v.at[b], gsem[b])

    def scatter(j, b):
        pltpu.async_copy(rows_v.at[b], acc.at[idx_d.at[j]],
                         ssem[b], add=True)

    def wait_scatter(b):
        pltpu.make_async_copy(rows_v.at[b], acc.at[pl.ds(0, CHB)],
                              ssem[b]).wait()

    def wait_gather(b):
        pltpu.make_async_copy(zero_hbm.at[pl.ds(0, CHB)], rows_v.at[b],
                              gsem[b]).wait()

    for b in range(3):
        gather(b, b)

    def body(g, carry):
        for b in range(_NBUF):
            j = g * _NBUF + b
            bn = (b + 3) % _NBUF

            @pl.when(j >= 2)
            def _():
                wait_scatter(bn)

            @pl.when(j < NCHB - 3)
            def _():
                gather(j + 3, bn)

            wait_gather(b)
            scatter(j, b)
        return carry

    lax.fori_loop(0, NCHB // _NBUF, body, 0)
    for j in (NCHB - 2, NCHB - 1):
        wait_scatter(j % _NBUF)
    plsc.subcore_barrier()
    pltpu.sync_copy(acc.at[pl.ds(sid * RPA, RPA)],
                    out_hbm.at[cid, pl.ds(sid * RPA, RPA)])


_BLK = 1280


def _prep_body(x_ref, w_ref, d0_ref, d1_ref, t_ref, dinv_ref):
    dp = d0_ref[...] + d1_ref[...]
    deg = 1.0 + jnp.sum(dp, axis=1, keepdims=True) * (1.0 / DW)
    dinv = lax.rsqrt(deg)
    xw = jnp.dot(x_ref[...], w_ref[...], preferred_element_type=jnp.float32)
    t_ref[...] = dinv * xw
    dinv_ref[...] = dinv


def _tc_prep(x, w1, d0, d1):
    return pl.pallas_call(
        _prep_body,
        grid=(NP // _BLK,),
        in_specs=[
            pl.BlockSpec((_BLK, D), lambda i: (i, 0)),
            pl.BlockSpec((D, D), lambda i: (0, 0)),
            pl.BlockSpec((_BLK, DW), lambda i: (i, 0)),
            pl.BlockSpec((_BLK, DW), lambda i: (i, 0)),
        ],
        out_specs=[
            pl.BlockSpec((_BLK, D), lambda i: (i, 0)),
            pl.BlockSpec((_BLK, 1), lambda i: (i, 0)),
        ],
        out_shape=[
            jax.ShapeDtypeStruct((NP, D), jnp.float32),
            jax.ShapeDtypeStruct((NP, 1), jnp.float32),
        ],
    )(x, w1, d0, d1)


def _mid_body(pa_ref, pb_ref, t_ref, dinv_ref, b_ref, w_ref, out_ref):
    s = jnp.concatenate([pa_ref[0], pb_ref[0]], axis=-1)
    dinv = dinv_ref[...]
    h = dinv * (s + t_ref[...]) + b_ref[...]
    h = jnp.maximum(h, 0.0)
    hw = jnp.dot(h, w_ref[...], preferred_element_type=jnp.float32)
    out_ref[...] = dinv * hw


def _tc_mid(p, t1, dinv, b1, w2):
    half = lambda k: pl.BlockSpec((1, _BLK, W), lambda i, k=k: (k, i, 0))
    return pl.pallas_call(
        _mid_body,
        grid=(NP // _BLK,),
        in_specs=[
            half(0), half(1),
            pl.BlockSpec((_BLK, D), lambda i: (i, 0)),
            pl.BlockSpec((_BLK, 1), lambda i: (i, 0)),
            pl.BlockSpec((1, D), lambda i: (0, 0)),
            pl.BlockSpec((D, D), lambda i: (0, 0)),
        ],
        out_specs=pl.BlockSpec((_BLK, D), lambda i: (i, 0)),
        out_shape=jax.ShapeDtypeStruct((NP, D), jnp.float32),
    )(p, p, t1, dinv, b1, w2)


def _out_body(qa_ref, qb_ref, t_ref, dinv_ref, b_ref, out_ref):
    s = jnp.concatenate([qa_ref[0], qb_ref[0]], axis=-1)
    out_ref[...] = dinv_ref[...] * (s + t_ref[...]) + b_ref[...]


def _tc_out(q, t2, dinv, b2):
    half = lambda k: pl.BlockSpec((1, _BLK, W), lambda i, k=k: (k, i, 0))
    return pl.pallas_call(
        _out_body,
        grid=(NP // _BLK,),
        in_specs=[
            half(0), half(1),
            pl.BlockSpec((_BLK, D), lambda i: (i, 0)),
            pl.BlockSpec((_BLK, 1), lambda i: (i, 0)),
            pl.BlockSpec((1, D), lambda i: (0, 0)),
        ],
        out_specs=pl.BlockSpec((_BLK, D), lambda i: (i, 0)),
        out_shape=jax.ShapeDtypeStruct((NP, D), jnp.float32),
    )(q, q, t2, dinv, b2)


def kernel(x, edge_index, W1, b1, W2, b2):
    ei = edge_index.astype(jnp.int32)
    padw = EPW * NW - E
    dstw = jnp.concatenate([ei[1], jnp.full((padw,), N, jnp.int32)]
                           ).reshape(NW, EPW)
    pads = EPS * NS - E
    srcs = jnp.concatenate([ei[0], jnp.zeros((pads,), jnp.int32)]
                           ).reshape(NS, NCHB, CHB)
    dsts = jnp.concatenate([ei[1], jnp.full((pads,), N, jnp.int32)]
                           ).reshape(NS, NCHB, CHB)
    x_p = jnp.pad(x, ((0, NP - N), (0, 0)))
    zero_w = jnp.zeros((RPS, W), jnp.float32)
    zero_deg = jnp.zeros((RPS, DW), jnp.float32)
    ones_deg = jnp.ones((CHD, DW), jnp.float32)

    degp = _sc_degree(dstw, zero_deg, ones_deg)
    t1, dinv = _tc_prep(x_p, W1, degp[0], degp[1])
    t1s = jnp.stack([t1[:, :W], t1[:, W:]])
    p = _sc_scatter(t1s, srcs, dsts, zero_w)
    t2 = _tc_mid(p, t1, dinv, b1.reshape(1, D), W2)
    t2s = jnp.stack([t2[:, :W], t2[:, W:]])
    q = _sc_scatter(t2s, srcs, dsts, zero_w)
    out = _tc_out(q, t2, dinv, b2.reshape(1, D))
    return out[:N]
